# baseline (speedup 1.0000x reference)
import jax
import jax.numpy as jnp
from jax import lax
from jax.experimental import pallas as pl
from jax.experimental.pallas import tpu as pltpu

_CONF_THRES = 0.001
_IOU_THRES = 0.45
_NC = 4
_MAX_WH = 4096.0
_MAX_DET = 300

_N = 5000
_NPAD = 5120
_ROWS = 40
_LANES = 128

_NF = 10


def _nms_body(x_ref, o_ref, f_ref):
    nb = x_ref.shape[0]

    def fld(c):
        return x_ref[:, c, :].reshape(-1, _ROWS, _LANES)

    cx, cy, w, h = fld(0), fld(1), fld(2), fld(3)
    obj = fld(4)
    x1 = cx - w / 2.0
    y1 = cy - h / 2.0
    x2 = cx + w / 2.0
    y2 = cy + h / 2.0

    c0 = fld(5) * obj
    c1 = fld(6) * obj
    c2 = fld(7) * obj
    c3 = fld(8) * obj
    conf = jnp.maximum(jnp.maximum(c0, c1), jnp.maximum(c2, c3))
    jf = jnp.where(
        c0 == conf,
        0.0,
        jnp.where(c1 == conf, 1.0, jnp.where(c2 == conf, 2.0, 3.0)),
    )

    iota0 = lax.broadcasted_iota(jnp.int32, (_ROWS, _LANES), 0)
    iota1 = lax.broadcasted_iota(jnp.int32, (_ROWS, _LANES), 1)
    idx0 = iota0 * _LANES + iota1
    scores0 = jnp.where((conf > _CONF_THRES) & (idx0 < _N)[None], conf, -1.0)

    off = jf * _MAX_WH
    x1o = x1 + off
    y1o = y1 + off
    x2o = x2 + off
    y2o = y2 + off
    areao = (x2o - x1o) * (y2o - y1o)

    for f, arr in enumerate([x1o, y1o, x2o, y2o, areao, x1, y1, x2, y2, jf]):
        f_ref[f] = arr.reshape(nb * _ROWS, _LANES)

    neg = jnp.float32(-jnp.inf)
    big = jnp.int32(1 << 30)

    def it(t, carry):
        lane = lax.broadcasted_iota(jnp.int32, (1, _LANES), 1)
        idx = lax.broadcasted_iota(jnp.int32, (_ROWS, _LANES), 0) * _LANES + lax.broadcasted_iota(
            jnp.int32, (_ROWS, _LANES), 1
        )
        new = []
        for b in range(nb):
            sc = carry[b]
            s = jnp.max(sc, axis=(0, 1), keepdims=True)
            m = jnp.min(jnp.where(sc == s, idx, big), axis=(0, 1), keepdims=True)
            sel = idx == m

            x1ob = f_ref[0, pl.ds(b * _ROWS, _ROWS), :]
            y1ob = f_ref[1, pl.ds(b * _ROWS, _ROWS), :]
            x2ob = f_ref[2, pl.ds(b * _ROWS, _ROWS), :]
            y2ob = f_ref[3, pl.ds(b * _ROWS, _ROWS), :]
            areaob = f_ref[4, pl.ds(b * _ROWS, _ROWS), :]

            def ext(arr):
                return jnp.sum(jnp.where(sel, arr, 0.0), axis=(0, 1), keepdims=True)

            bx1o = ext(x1ob)
            by1o = ext(y1ob)
            bx2o = ext(x2ob)
            by2o = ext(y2ob)
            barea = ext(areaob)

            ltx = jnp.maximum(bx1o, x1ob)
            lty = jnp.maximum(by1o, y1ob)
            rbx = jnp.minimum(bx2o, x2ob)
            rby = jnp.minimum(by2o, y2ob)
            iw = jnp.clip(rbx - ltx, 0.0, None)
            ih = jnp.clip(rby - lty, 0.0, None)
            inter = iw * ih
            iou = inter / (barea + areaob - inter + 1e-9)

            kill = (iou > _IOU_THRES) | sel
            new.append(jnp.where(kill, neg, sc))

            emit = s > _CONF_THRES
            row = jnp.where(
                lane == 0,
                ext(f_ref[5, pl.ds(b * _ROWS, _ROWS), :]),
                jnp.where(
                    lane == 1,
                    ext(f_ref[6, pl.ds(b * _ROWS, _ROWS), :]),
                    jnp.where(
                        lane == 2,
                        ext(f_ref[7, pl.ds(b * _ROWS, _ROWS), :]),
                        jnp.where(
                            lane == 3,
                            ext(f_ref[8, pl.ds(b * _ROWS, _ROWS), :]),
                            jnp.where(
                                lane == 4,
                                s,
                                jnp.where(
                                    lane == 5,
                                    ext(f_ref[9, pl.ds(b * _ROWS, _ROWS), :]),
                                    0.0,
                                ),
                            ),
                        ),
                    ),
                ),
            )
            o_ref[b, pl.ds(t, 1), :] = jnp.where(emit, row, 0.0)
        return tuple(new)

    lax.fori_loop(0, _MAX_DET, it, tuple(scores0[b] for b in range(nb)))


def kernel(x):
    pred = x[0]
    b = pred.shape[0]
    predt = jnp.transpose(pred, (0, 2, 1))
    predt = jnp.pad(predt, ((0, 0), (0, 0), (0, _NPAD - _N)))

    out = pl.pallas_call(
        _nms_body,
        out_shape=jax.ShapeDtypeStruct((b, _MAX_DET, _LANES), jnp.float32),
        scratch_shapes=[pltpu.VMEM((_NF, b * _ROWS, _LANES), jnp.float32)],
    )(predt)
    return out[:, :, :6]


# speedup vs baseline: 1.6291x; 1.6291x over previous
import jax
import jax.numpy as jnp
from jax import lax
from jax.experimental import pallas as pl
from jax.experimental.pallas import tpu as pltpu

_CONF_THRES = 0.001
_IOU_THRES = 0.45
_NC = 4
_MAX_WH = 4096.0
_MAX_DET = 300

_N = 5000
_NPAD = 5120
_ROWS = 40
_LANES = 128

_NF = 10


def _nms_body(x_ref, o_ref, f_ref):
    nb = x_ref.shape[0]

    def fld(c):
        return x_ref[:, c, :].reshape(-1, _ROWS, _LANES)

    cx, cy, w, h = fld(0), fld(1), fld(2), fld(3)
    obj = fld(4)
    x1 = cx - w / 2.0
    y1 = cy - h / 2.0
    x2 = cx + w / 2.0
    y2 = cy + h / 2.0

    c0 = fld(5) * obj
    c1 = fld(6) * obj
    c2 = fld(7) * obj
    c3 = fld(8) * obj
    conf = jnp.maximum(jnp.maximum(c0, c1), jnp.maximum(c2, c3))
    jf = jnp.where(
        c0 == conf,
        0.0,
        jnp.where(c1 == conf, 1.0, jnp.where(c2 == conf, 2.0, 3.0)),
    )

    iota0 = lax.broadcasted_iota(jnp.int32, (_ROWS, _LANES), 0)
    iota1 = lax.broadcasted_iota(jnp.int32, (_ROWS, _LANES), 1)
    idx0 = iota0 * _LANES + iota1
    scores0 = jnp.where((conf > _CONF_THRES) & (idx0 < _N)[None], conf, -1.0)

    off = jf * _MAX_WH
    x1o = x1 + off
    y1o = y1 + off
    x2o = x2 + off
    y2o = y2 + off
    areao = (x2o - x1o) * (y2o - y1o)

    for f, arr in enumerate([x1o, y1o, x2o, y2o, areao, x1, y1, x2, y2, jf]):
        f_ref[f] = arr.reshape(nb * _ROWS, _LANES)

    neg = jnp.float32(-jnp.inf)
    big = jnp.int32(1 << 30)

    def it(t, carry):
        lane = lax.broadcasted_iota(jnp.int32, (1, _LANES), 1)
        idx = lax.broadcasted_iota(jnp.int32, (_ROWS, _LANES), 0) * _LANES + lax.broadcasted_iota(
            jnp.int32, (_ROWS, _LANES), 1
        )
        s = [jnp.max(carry[b], axis=(0, 1), keepdims=True) for b in range(nb)]
        m = [
            jnp.min(jnp.where(carry[b] == s[b], idx, big), axis=(0, 1), keepdims=True)
            for b in range(nb)
        ]
        sel = [idx == m[b] for b in range(nb)]

        def fr(f, b):
            return f_ref[f, pl.ds(b * _ROWS, _ROWS), :]

        def ext(arr, b):
            return jnp.sum(jnp.where(sel[b], arr, 0.0), axis=(0, 1), keepdims=True)

        ex = [[ext(fr(f, b), b) for f in range(_NF)] for b in range(nb)]

        new = []
        for b in range(nb):
            x1ob, y1ob, x2ob, y2ob, areaob = (fr(f, b) for f in range(5))
            bx1o, by1o, bx2o, by2o, barea = ex[b][:5]
            ltx = jnp.maximum(bx1o, x1ob)
            lty = jnp.maximum(by1o, y1ob)
            rbx = jnp.minimum(bx2o, x2ob)
            rby = jnp.minimum(by2o, y2ob)
            iw = jnp.clip(rbx - ltx, 0.0, None)
            ih = jnp.clip(rby - lty, 0.0, None)
            inter = iw * ih
            iou = inter / (barea + areaob - inter + 1e-9)

            kill = (iou > _IOU_THRES) | sel[b]
            new.append(jnp.where(kill, neg, carry[b]))

        for b in range(nb):
            emit = s[b] > _CONF_THRES
            row = jnp.where(
                lane == 0,
                ex[b][5],
                jnp.where(
                    lane == 1,
                    ex[b][6],
                    jnp.where(
                        lane == 2,
                        ex[b][7],
                        jnp.where(
                            lane == 3,
                            ex[b][8],
                            jnp.where(
                                lane == 4,
                                s[b],
                                jnp.where(lane == 5, ex[b][9], 0.0),
                            ),
                        ),
                    ),
                ),
            )
            o_ref[b, pl.ds(t, 1), :] = jnp.where(emit, row, 0.0)
        return tuple(new)

    lax.fori_loop(0, _MAX_DET, it, tuple(scores0[b] for b in range(nb)))


def kernel(x):
    pred = x[0]
    b = pred.shape[0]
    predt = jnp.transpose(pred, (0, 2, 1))
    predt = jnp.pad(predt, ((0, 0), (0, 0), (0, _NPAD - _N)))

    out = pl.pallas_call(
        _nms_body,
        out_shape=jax.ShapeDtypeStruct((b, _MAX_DET, _LANES), jnp.float32),
        scratch_shapes=[pltpu.VMEM((_NF, b * _ROWS, _LANES), jnp.float32)],
    )(predt)
    return out[:, :, :6]


# speedup vs baseline: 1.7308x; 1.0624x over previous
import jax
import jax.numpy as jnp
from jax import lax
from jax.experimental import pallas as pl
from jax.experimental.pallas import tpu as pltpu

_CONF_THRES = 0.001
_IOU_THRES = 0.45
_NC = 4
_MAX_WH = 4096.0
_MAX_DET = 300

_N = 5000
_NPAD = 5120
_ROWS = 40
_LANES = 128

_NF = 11


def _nms_body(x_ref, o_ref, f_ref):
    nb = x_ref.shape[0]

    def fld(c):
        return x_ref[:, c, :].reshape(-1, _ROWS, _LANES)

    cx, cy, w, h = fld(0), fld(1), fld(2), fld(3)
    obj = fld(4)
    x1 = cx - w / 2.0
    y1 = cy - h / 2.0
    x2 = cx + w / 2.0
    y2 = cy + h / 2.0

    c0 = fld(5) * obj
    c1 = fld(6) * obj
    c2 = fld(7) * obj
    c3 = fld(8) * obj
    conf = jnp.maximum(jnp.maximum(c0, c1), jnp.maximum(c2, c3))
    jf = jnp.where(
        c0 == conf,
        0.0,
        jnp.where(c1 == conf, 1.0, jnp.where(c2 == conf, 2.0, 3.0)),
    )

    iota0 = lax.broadcasted_iota(jnp.int32, (_ROWS, _LANES), 0)
    iota1 = lax.broadcasted_iota(jnp.int32, (_ROWS, _LANES), 1)
    idx0 = iota0 * _LANES + iota1
    scores0 = jnp.where((conf > _CONF_THRES) & (idx0 < _N)[None], conf, -1.0)

    off = jf * _MAX_WH
    x1o = x1 + off
    y1o = y1 + off
    x2o = x2 + off
    y2o = y2 + off
    areao = (x2o - x1o) * (y2o - y1o)

    for f, arr in enumerate([x1o, y1o, x2o, y2o, areao, x1, y1, x2, y2, jf, conf]):
        f_ref[f] = arr.reshape(nb * _ROWS, _LANES)

    neg = jnp.float32(-jnp.inf)
    li = lax.broadcasted_iota(jnp.int32, (_LANES, _LANES), 0)
    lj = lax.broadcasted_iota(jnp.int32, (_LANES, _LANES), 1)

    _OFF = jnp.int32(0x3A800000)
    _MASK27 = jnp.int32(0x07FFFFFF)
    _SIGN = jnp.int32(-(1 << 31))

    def it(t, carry):
        lane = lax.broadcasted_iota(jnp.int32, (1, _LANES), 1)
        rowc = 39 - lax.broadcasted_iota(jnp.int32, (_ROWS, _LANES), 0)
        tri = (li < lj).astype(jnp.float32)
        keys = []
        for b in range(nb):
            sc = carry[b]
            bits = lax.bitcast_convert_type(sc, jnp.int32)
            valid = sc > _CONF_THRES
            b27 = bits - _OFF
            kb = jnp.where(valid, (b27 << 5) + (b27 << 3) + rowc, rowc) ^ _SIGN
            keys.append(kb)
        kcol = [jnp.max(keys[b], axis=0, keepdims=True) for b in range(nb)]
        kstar = [jnp.max(kcol[b], axis=1, keepdims=True) for b in range(nb)]
        winners = [kcol[b] == kstar[b] for b in range(nb)]
        prefix = [
            jnp.dot(winners[b].astype(jnp.float32), tri) for b in range(nb)
        ]
        sel_lane = [winners[b] & (prefix[b] == 0.0) for b in range(nb)]
        sel = [(keys[b] == kstar[b]) & sel_lane[b] for b in range(nb)]
        emit = [kstar[b] > jnp.int32(-(1 << 31) + 39) for b in range(nb)]

        def fr(f, b):
            return f_ref[f, pl.ds(b * _ROWS, _ROWS), :]

        def ext(arr, b):
            return jnp.sum(jnp.where(sel[b], arr, 0.0), axis=(0, 1), keepdims=True)

        ex = [[ext(fr(f, b), b) for f in range(_NF)] for b in range(nb)]

        new = []
        for b in range(nb):
            x1ob, y1ob, x2ob, y2ob, areaob = (fr(f, b) for f in range(5))
            bx1o, by1o, bx2o, by2o, barea = ex[b][:5]
            ltx = jnp.maximum(bx1o, x1ob)
            lty = jnp.maximum(by1o, y1ob)
            rbx = jnp.minimum(bx2o, x2ob)
            rby = jnp.minimum(by2o, y2ob)
            iw = jnp.clip(rbx - ltx, 0.0, None)
            ih = jnp.clip(rby - lty, 0.0, None)
            inter = iw * ih
            iou = inter / (barea + areaob - inter + 1e-9)

            kill = (iou > _IOU_THRES) | sel[b]
            new.append(jnp.where(kill, neg, carry[b]))

        for b in range(nb):
            row = jnp.where(
                lane == 0,
                ex[b][5],
                jnp.where(
                    lane == 1,
                    ex[b][6],
                    jnp.where(
                        lane == 2,
                        ex[b][7],
                        jnp.where(
                            lane == 3,
                            ex[b][8],
                            jnp.where(
                                lane == 4,
                                ex[b][10],
                                jnp.where(lane == 5, ex[b][9], 0.0),
                            ),
                        ),
                    ),
                ),
            )
            o_ref[b, pl.ds(t, 1), :] = jnp.where(emit[b], row, 0.0)
        return tuple(new)

    lax.fori_loop(0, _MAX_DET, it, tuple(scores0[b] for b in range(nb)))


def kernel(x):
    pred = x[0]
    b = pred.shape[0]
    predt = jnp.transpose(pred, (0, 2, 1))
    predt = jnp.pad(predt, ((0, 0), (0, 0), (0, _NPAD - _N)))

    out = pl.pallas_call(
        _nms_body,
        out_shape=jax.ShapeDtypeStruct((b, _MAX_DET, _LANES), jnp.float32),
        scratch_shapes=[pltpu.VMEM((_NF, b * _ROWS, _LANES), jnp.float32)],
    )(predt)
    return out[:, :, :6]


# speedup vs baseline: 1.9596x; 1.1322x over previous
import jax
import jax.numpy as jnp
from jax import lax
from jax.experimental import pallas as pl
from jax.experimental.pallas import tpu as pltpu

_CONF_THRES = 0.001
_IOU_THRES = 0.45
_NC = 4
_MAX_WH = 4096.0
_MAX_DET = 300

_N = 5000
_NPAD = 5120
_ROWS = 40
_LANES = 128

_NF = 10


def _nms_body(x_ref, o_ref, f_ref):
    nb = x_ref.shape[0]

    def fld(c):
        return x_ref[:, c, :].reshape(-1, _ROWS, _LANES)

    cx, cy, w, h = fld(0), fld(1), fld(2), fld(3)
    obj = fld(4)
    x1 = cx - w / 2.0
    y1 = cy - h / 2.0
    x2 = cx + w / 2.0
    y2 = cy + h / 2.0

    c0 = fld(5) * obj
    c1 = fld(6) * obj
    c2 = fld(7) * obj
    c3 = fld(8) * obj
    conf = jnp.maximum(jnp.maximum(c0, c1), jnp.maximum(c2, c3))
    jf = jnp.where(
        c0 == conf,
        0.0,
        jnp.where(c1 == conf, 1.0, jnp.where(c2 == conf, 2.0, 3.0)),
    )

    iota0 = lax.broadcasted_iota(jnp.int32, (_ROWS, _LANES), 0)
    iota1 = lax.broadcasted_iota(jnp.int32, (_ROWS, _LANES), 1)
    idx0 = iota0 * _LANES + iota1
    scores0 = jnp.where((conf > _CONF_THRES) & (idx0 < _N)[None], conf, -1.0)

    off = jf * _MAX_WH
    x1o = x1 + off
    y1o = y1 + off
    x2o = x2 + off
    y2o = y2 + off
    areao = (x2o - x1o) * (y2o - y1o)

    for f, arr in enumerate([x1o, y1o, x2o, y2o, areao, x1, y1, x2, y2, jf]):
        f_ref[f] = arr.reshape(nb * _ROWS, _LANES)

    neg = jnp.float32(-jnp.inf)

    def it(t, carry):
        lane = lax.broadcasted_iota(jnp.int32, (1, _LANES), 1)
        idxf = (
            lax.broadcasted_iota(jnp.int32, (_ROWS, _LANES), 0) * _LANES
            + lax.broadcasted_iota(jnp.int32, (_ROWS, _LANES), 1)
        ).astype(jnp.float32)
        s = [jnp.max(carry[b], axis=(0, 1), keepdims=True) for b in range(nb)]
        key = [
            jnp.where(carry[b] == s[b], idxf, jnp.float32(1e9)) for b in range(nb)
        ]
        m = [jnp.min(key[b], axis=(0, 1), keepdims=True) for b in range(nb)]
        sel = [key[b] == m[b] for b in range(nb)]
        emit = [s[b] > _CONF_THRES for b in range(nb)]

        def fr(f, b):
            return f_ref[f, pl.ds(b * _ROWS, _ROWS), :]

        def ext(arr, b):
            return jnp.sum(jnp.where(sel[b], arr, 0.0), axis=(0, 1), keepdims=True)

        ex = [[ext(fr(f, b), b) for f in range(_NF)] for b in range(nb)]

        new = []
        for b in range(nb):
            x1ob, y1ob, x2ob, y2ob, areaob = (fr(f, b) for f in range(5))
            bx1o, by1o, bx2o, by2o, barea = ex[b][:5]
            ltx = jnp.maximum(bx1o, x1ob)
            lty = jnp.maximum(by1o, y1ob)
            rbx = jnp.minimum(bx2o, x2ob)
            rby = jnp.minimum(by2o, y2ob)
            iw = jnp.clip(rbx - ltx, 0.0, None)
            ih = jnp.clip(rby - lty, 0.0, None)
            inter = iw * ih
            iou = inter / (barea + areaob - inter + 1e-9)

            kill = (iou > _IOU_THRES) | sel[b]
            new.append(jnp.where(kill, neg, carry[b]))

        for b in range(nb):
            row = jnp.where(
                lane == 0,
                ex[b][5],
                jnp.where(
                    lane == 1,
                    ex[b][6],
                    jnp.where(
                        lane == 2,
                        ex[b][7],
                        jnp.where(
                            lane == 3,
                            ex[b][8],
                            jnp.where(
                                lane == 4,
                                s[b],
                                jnp.where(lane == 5, ex[b][9], 0.0),
                            ),
                        ),
                    ),
                ),
            )
            o_ref[b, pl.ds(t, 1), :] = jnp.where(emit[b], row, 0.0)
        return tuple(new)

    lax.fori_loop(0, _MAX_DET, it, tuple(scores0[b] for b in range(nb)))


def kernel(x):
    pred = x[0]
    b = pred.shape[0]
    predt = jnp.transpose(pred, (0, 2, 1))
    predt = jnp.pad(predt, ((0, 0), (0, 0), (0, _NPAD - _N)))

    out = pl.pallas_call(
        _nms_body,
        out_shape=jax.ShapeDtypeStruct((b, _MAX_DET, _LANES), jnp.float32),
        scratch_shapes=[pltpu.VMEM((_NF, b * _ROWS, _LANES), jnp.float32)],
    )(predt)
    return out[:, :, :6]


# speedup vs baseline: 2.4764x; 1.2638x over previous
import jax
import jax.numpy as jnp
from jax import lax
from jax.experimental import pallas as pl
from jax.experimental.pallas import tpu as pltpu

_CONF_THRES = 0.001
_IOU_THRES = 0.45
_NC = 4
_MAX_WH = 4096.0
_MAX_DET = 300

_N = 5000
_NPAD = 5120
_ROWS = 40
_LANES = 128

_NF = 10


def _nms_body(x_ref, o_ref, f_ref, s_ref):
    nb = x_ref.shape[0]

    def fld(c):
        return x_ref[:, c, :].reshape(-1, _ROWS, _LANES)

    cx, cy, w, h = fld(0), fld(1), fld(2), fld(3)
    obj = fld(4)
    x1 = cx - w / 2.0
    y1 = cy - h / 2.0
    x2 = cx + w / 2.0
    y2 = cy + h / 2.0

    c0 = fld(5) * obj
    c1 = fld(6) * obj
    c2 = fld(7) * obj
    c3 = fld(8) * obj
    conf = jnp.maximum(jnp.maximum(c0, c1), jnp.maximum(c2, c3))
    jf = jnp.where(
        c0 == conf,
        0.0,
        jnp.where(c1 == conf, 1.0, jnp.where(c2 == conf, 2.0, 3.0)),
    )

    iota0 = lax.broadcasted_iota(jnp.int32, (_ROWS, _LANES), 0)
    iota1 = lax.broadcasted_iota(jnp.int32, (_ROWS, _LANES), 1)
    idx0 = iota0 * _LANES + iota1
    scores0 = jnp.where((conf > _CONF_THRES) & (idx0 < _N)[None], conf, -1.0)

    off = jf * _MAX_WH
    x1o = x1 + off
    y1o = y1 + off
    x2o = x2 + off
    y2o = y2 + off
    areao = (x2o - x1o) * (y2o - y1o)

    for f, arr in enumerate([x1o, y1o, x2o, y2o, areao, x1, y1, x2, y2, jf]):
        f_ref[f] = arr.reshape(nb * _ROWS, _LANES)

    neg = jnp.float32(-jnp.inf)

    s_ref[...] = scores0.reshape(nb * _ROWS, _LANES)

    def it(t, carry):
        lane = lax.broadcasted_iota(jnp.int32, (1, _LANES), 1)
        idxf = (
            lax.broadcasted_iota(jnp.int32, (_ROWS, _LANES), 0) * _LANES
            + lax.broadcasted_iota(jnp.int32, (_ROWS, _LANES), 1)
        ).astype(jnp.float32)

        def fr(f, b):
            return f_ref[f, pl.ds(b * _ROWS, _ROWS), :]

        sc = [s_ref[pl.ds(b * _ROWS, _ROWS), :] for b in range(nb)]
        s = [jnp.max(sc[b], axis=(0, 1), keepdims=True) for b in range(nb)]
        wm = [sc[b] == s[b] for b in range(nb)]
        cnt = [
            jnp.sum(wm[b].astype(jnp.float32), axis=(0, 1), keepdims=True)
            for b in range(nb)
        ]
        emit = [s[b] > _CONF_THRES for b in range(nb)]

        def extw(arr, b):
            return jnp.sum(jnp.where(wm[b], arr, 0.0), axis=(0, 1), keepdims=True)

        ex = [[extw(fr(f, b), b) for f in range(_NF)] for b in range(nb)]

        def finish(b, sel, exb):
            x1ob, y1ob, x2ob, y2ob, areaob = (fr(f, b) for f in range(5))
            bx1o, by1o, bx2o, by2o, barea = exb[:5]
            ltx = jnp.maximum(bx1o, x1ob)
            lty = jnp.maximum(by1o, y1ob)
            rbx = jnp.minimum(bx2o, x2ob)
            rby = jnp.minimum(by2o, y2ob)
            iw = jnp.clip(rbx - ltx, 0.0, None)
            ih = jnp.clip(rby - lty, 0.0, None)
            inter = iw * ih
            iou = inter / (barea + areaob - inter + 1e-9)
            kill = (iou > _IOU_THRES) | sel
            s_ref[pl.ds(b * _ROWS, _ROWS), :] = jnp.where(kill, neg, sc[b])
            row = jnp.where(
                lane == 0,
                exb[5],
                jnp.where(
                    lane == 1,
                    exb[6],
                    jnp.where(
                        lane == 2,
                        exb[7],
                        jnp.where(
                            lane == 3,
                            exb[8],
                            jnp.where(
                                lane == 4,
                                s[b],
                                jnp.where(lane == 5, exb[9], 0.0),
                            ),
                        ),
                    ),
                ),
            )
            o_ref[b, pl.ds(t, 1), :] = jnp.where(emit[b], row, 0.0)

        for b in range(nb):
            unique = cnt[b][0, 0] < 1.5

            @pl.when(unique)
            def _():
                finish(b, wm[b], ex[b])

            @pl.when(jnp.logical_not(unique))
            def _():
                key = jnp.where(wm[b], idxf, jnp.float32(1e9))
                m = jnp.min(key, axis=(0, 1), keepdims=True)
                sel = key == m
                exb = [
                    jnp.sum(jnp.where(sel, fr(f, b), 0.0), axis=(0, 1), keepdims=True)
                    for f in range(_NF)
                ]
                finish(b, sel, exb)

        return carry

    lax.fori_loop(0, _MAX_DET, it, 0)


def kernel(x):
    pred = x[0]
    b = pred.shape[0]
    predt = jnp.transpose(pred, (0, 2, 1))
    predt = jnp.pad(predt, ((0, 0), (0, 0), (0, _NPAD - _N)))

    out = pl.pallas_call(
        _nms_body,
        out_shape=jax.ShapeDtypeStruct((b, _MAX_DET, _LANES), jnp.float32),
        scratch_shapes=[
            pltpu.VMEM((_NF, b * _ROWS, _LANES), jnp.float32),
            pltpu.VMEM((b * _ROWS, _LANES), jnp.float32),
        ],
    )(predt)
    return out[:, :, :6]


# speedup vs baseline: 2.4827x; 1.0025x over previous
import jax
import jax.numpy as jnp
from jax import lax
from jax.experimental import pallas as pl
from jax.experimental.pallas import tpu as pltpu

_CONF_THRES = 0.001
_IOU_THRES = 0.45
_NC = 4
_MAX_WH = 4096.0
_MAX_DET = 300

_N = 5000
_NPAD = 5120
_ROWS = 40
_LANES = 128

_NF = 10


def _nms_body(x_ref, o_ref, f_ref, s_ref):
    nb = x_ref.shape[0]

    def fld(c):
        return x_ref[:, c, :].reshape(-1, _ROWS, _LANES)

    cx, cy, w, h = fld(0), fld(1), fld(2), fld(3)
    obj = fld(4)
    x1 = cx - w / 2.0
    y1 = cy - h / 2.0
    x2 = cx + w / 2.0
    y2 = cy + h / 2.0

    c0 = fld(5) * obj
    c1 = fld(6) * obj
    c2 = fld(7) * obj
    c3 = fld(8) * obj
    conf = jnp.maximum(jnp.maximum(c0, c1), jnp.maximum(c2, c3))
    jf = jnp.where(
        c0 == conf,
        0.0,
        jnp.where(c1 == conf, 1.0, jnp.where(c2 == conf, 2.0, 3.0)),
    )

    iota0 = lax.broadcasted_iota(jnp.int32, (_ROWS, _LANES), 0)
    iota1 = lax.broadcasted_iota(jnp.int32, (_ROWS, _LANES), 1)
    idx0 = iota0 * _LANES + iota1
    scores0 = jnp.where((conf > _CONF_THRES) & (idx0 < _N)[None], conf, -1.0)

    off = jf * _MAX_WH
    x1o = x1 + off
    y1o = y1 + off
    x2o = x2 + off
    y2o = y2 + off
    areao = (x2o - x1o) * (y2o - y1o)

    for f, arr in enumerate([x1o, y1o, x2o, y2o, areao, x1, y1, x2, y2, jf]):
        f_ref[f] = arr.reshape(nb * _ROWS, _LANES)

    neg = jnp.float32(-jnp.inf)

    s_ref[...] = scores0.reshape(nb * _ROWS, _LANES)

    def it(t, carry):
        lane = lax.broadcasted_iota(jnp.int32, (1, _LANES), 1)

        def fr(f, b):
            return f_ref[f, pl.ds(b * _ROWS, _ROWS), :]

        sc = [s_ref[pl.ds(b * _ROWS, _ROWS), :] for b in range(nb)]
        s = [jnp.max(sc[b], axis=(0, 1), keepdims=True) for b in range(nb)]
        wm = [sc[b] == s[b] for b in range(nb)]
        cnt = [
            jnp.sum(wm[b].astype(jnp.float32), axis=(0, 1), keepdims=True)
            for b in range(nb)
        ]
        emit = [s[b] > _CONF_THRES for b in range(nb)]

        def extw(arr, b):
            return jnp.sum(jnp.where(wm[b], arr, 0.0), axis=(0, 1), keepdims=True)

        ex = [[extw(fr(f, b), b) for f in range(_NF)] for b in range(nb)]

        def finish(b, sel, exb):
            x1ob, y1ob, x2ob, y2ob, areaob = (fr(f, b) for f in range(5))
            bx1o, by1o, bx2o, by2o, barea = exb[:5]
            ltx = jnp.maximum(bx1o, x1ob)
            lty = jnp.maximum(by1o, y1ob)
            rbx = jnp.minimum(bx2o, x2ob)
            rby = jnp.minimum(by2o, y2ob)
            iw = jnp.clip(rbx - ltx, 0.0, None)
            ih = jnp.clip(rby - lty, 0.0, None)
            inter = iw * ih
            iou = inter / (barea + areaob - inter + 1e-9)
            kill = (iou > _IOU_THRES) | sel
            s_ref[pl.ds(b * _ROWS, _ROWS), :] = jnp.where(kill, neg, sc[b])
            row = jnp.where(
                lane == 0,
                exb[5],
                jnp.where(
                    lane == 1,
                    exb[6],
                    jnp.where(
                        lane == 2,
                        exb[7],
                        jnp.where(
                            lane == 3,
                            exb[8],
                            jnp.where(
                                lane == 4,
                                s[b],
                                jnp.where(lane == 5, exb[9], 0.0),
                            ),
                        ),
                    ),
                ),
            )
            o_ref[b, pl.ds(t, 1), :] = jnp.where(emit[b], row, 0.0)

        for b in range(nb):
            unique = cnt[b][0, 0] < 1.5

            @pl.when(unique)
            def _():
                finish(b, wm[b], ex[b])

            @pl.when(jnp.logical_not(unique))
            def _():
                idxf = (
                    lax.broadcasted_iota(jnp.int32, (_ROWS, _LANES), 0) * _LANES
                    + lax.broadcasted_iota(jnp.int32, (_ROWS, _LANES), 1)
                ).astype(jnp.float32)
                key = jnp.where(wm[b], idxf, jnp.float32(1e9))
                m = jnp.min(key, axis=(0, 1), keepdims=True)
                sel = key == m
                exb = [
                    jnp.sum(jnp.where(sel, fr(f, b), 0.0), axis=(0, 1), keepdims=True)
                    for f in range(_NF)
                ]
                finish(b, sel, exb)

        return carry

    lax.fori_loop(0, _MAX_DET, it, 0)


def kernel(x):
    pred = x[0]
    b = pred.shape[0]
    predt = jnp.transpose(pred, (0, 2, 1))
    predt = jnp.pad(predt, ((0, 0), (0, 0), (0, _NPAD - _N)))

    out = pl.pallas_call(
        _nms_body,
        out_shape=jax.ShapeDtypeStruct((b, _MAX_DET, _LANES), jnp.float32),
        scratch_shapes=[
            pltpu.VMEM((_NF, b * _ROWS, _LANES), jnp.float32),
            pltpu.VMEM((b * _ROWS, _LANES), jnp.float32),
        ],
    )(predt)
    return out[:, :, :6]
